# Initial kernel scaffold; baseline (speedup 1.0000x reference)
import functools

import jax
import jax.numpy as jnp
from jax import lax
from jax.experimental import pallas as pl
from jax.experimental.pallas import tpu as pltpu
from jax.experimental.pallas import tpu_sc as plsc

B = 16384
S = 20
D = 32
HIDDEN = 256

NC, NS = 2, 16
NW = NC * NS
BPW = B // NW
CHUNK = 128
NCHUNK = BPW // CHUNK


def _gather_sum_body(idx_hbm, table_hbm, out_hbm, idx_v, acc_v, sem):
    wid = lax.axis_index("s") * NC + lax.axis_index("c")
    base = wid * BPW

    pltpu.sync_copy(idx_hbm.at[wid], idx_v)

    zero = jnp.zeros((16,), jnp.float32)

    def zero_body(i, carry):
        for u in range(4):
            r = i * 4 + u
            acc_v[r, pl.ds(0, 16)] = zero
            acc_v[r, pl.ds(16, 16)] = zero
        return carry

    lax.fori_loop(0, BPW // 4, zero_body, 0)

    def chunk_body(c, carry):
        dst = acc_v.at[pl.ds(c * CHUNK, CHUNK)]
        descs = []
        for j in range(S):
            d = pltpu.async_copy(
                table_hbm.at[idx_v.at[c * S + j]], dst, sem, add=True
            )
            descs.append(d)
        for d in descs:
            d.wait()
        return carry

    lax.fori_loop(0, NCHUNK, chunk_body, 0)

    pltpu.sync_copy(acc_v, out_hbm.at[pl.ds(base, BPW)])


def _gather_sum(idx3, table):
    mesh = plsc.VectorSubcoreMesh(
        core_axis_name="c", subcore_axis_name="s", num_cores=NC, num_subcores=NS
    )
    return pl.kernel(
        _gather_sum_body,
        out_type=jax.ShapeDtypeStruct((B, D), jnp.float32),
        mesh=mesh,
        scratch_types=[
            pltpu.VMEM((NCHUNK * S, CHUNK), jnp.int32),
            pltpu.VMEM((BPW, D), jnp.float32),
            pltpu.SemaphoreType.DMA,
        ],
    )(idx3, table)


def _mlp_body(x_ref, flag_ref, w1t_ref, w1f_ref, b1_ref, w2_ref, b2_ref, o_ref):
    x = x_ref[...]
    h = jnp.dot(x, w1t_ref[...], preferred_element_type=jnp.float32)
    h = h + flag_ref[...] * w1f_ref[...] + b1_ref[...]
    h = jnp.maximum(h, 0.0)
    o = jnp.dot(h, w2_ref[...], preferred_element_type=jnp.float32)
    o_ref[...] = o + b2_ref[0, 0]


def _mlp(summed, flag, w1t, w1f, b1_2d, w2t, b2_2d):
    BT = 2048
    grid = (B // BT,)
    return pl.pallas_call(
        _mlp_body,
        grid=grid,
        in_specs=[
            pl.BlockSpec((BT, D), lambda i: (i, 0)),
            pl.BlockSpec((BT, 1), lambda i: (i, 0)),
            pl.BlockSpec((D, HIDDEN), lambda i: (0, 0)),
            pl.BlockSpec((1, HIDDEN), lambda i: (0, 0)),
            pl.BlockSpec((1, HIDDEN), lambda i: (0, 0)),
            pl.BlockSpec((HIDDEN, 1), lambda i: (0, 0)),
            pl.BlockSpec((1, 1), lambda i: (0, 0)),
        ],
        out_specs=pl.BlockSpec((BT, 1), lambda i: (i, 0)),
        out_shape=jax.ShapeDtypeStruct((B, 1), jnp.float32),
    )(summed, flag, w1t, w1f, b1_2d, w2t, b2_2d)


def kernel(lineup, table, W1, b1, W2, b2):
    idx3 = (
        lineup[:, :S]
        .reshape(NW, NCHUNK, CHUNK, S)
        .transpose(0, 1, 3, 2)
        .reshape(NW, NCHUNK * S, CHUNK)
    )
    summed = _gather_sum(idx3, table)

    flag = lineup[:, S:].astype(jnp.float32)
    w1t = W1[:, :D].T
    w1f = W1[:, D].reshape(1, HIDDEN)
    return _mlp(summed, flag, w1t, w1f, b1.reshape(1, HIDDEN), W2.T,
                b2.reshape(1, 1))


# speedup vs baseline: 1.7054x; 1.7054x over previous
import functools

import jax
import jax.numpy as jnp
from jax import lax
from jax.experimental import pallas as pl
from jax.experimental.pallas import tpu as pltpu
from jax.experimental.pallas import tpu_sc as plsc

B = 16384
S = 20
D = 32
HIDDEN = 256

NC, NS = 2, 16
NW = NC * NS
BPW = B // NW
CHUNK = 128
NCHUNK = BPW // CHUNK


def _gather_sum_body(idx_hbm, table_hbm, out_hbm, idx_v, acc_v, sem):
    wid = lax.axis_index("s") * NC + lax.axis_index("c")
    base = wid * BPW

    pltpu.sync_copy(idx_hbm.at[wid], idx_v)

    zero = jnp.zeros((16,), jnp.float32)

    def zero_body(i, carry):
        for u in range(4):
            r = i * 4 + u
            acc_v[r, pl.ds(0, 16)] = zero
            acc_v[r, pl.ds(16, 16)] = zero
        return carry

    lax.fori_loop(0, BPW // 4, zero_body, 0)

    def chunk_body(c, carry):
        dst = acc_v.at[pl.ds(c * CHUNK, CHUNK)]
        descs = []
        for j in range(S):
            d = pltpu.async_copy(
                table_hbm.at[idx_v.at[c * S + j]], dst, sem, add=True
            )
            descs.append(d)
        for d in descs:
            d.wait()
        return carry

    lax.fori_loop(0, NCHUNK, chunk_body, 0)

    pltpu.sync_copy(acc_v, out_hbm.at[pl.ds(base, BPW)])


def _gather_sum(idx3, table):
    mesh = plsc.VectorSubcoreMesh(
        core_axis_name="c", subcore_axis_name="s", num_cores=NC, num_subcores=NS
    )
    return pl.kernel(
        _gather_sum_body,
        out_type=jax.ShapeDtypeStruct((B, D), jnp.float32),
        mesh=mesh,
        scratch_types=[
            pltpu.VMEM((NCHUNK * S, CHUNK), jnp.int32),
            pltpu.VMEM((BPW, D), jnp.float32),
            pltpu.SemaphoreType.DMA,
        ],
        compiler_params=pltpu.CompilerParams(use_tc_tiling_on_sc=False),
    )(idx3, table)


def _mlp_body(x_ref, w1t_ref, b1_ref, w2_ref, b2_ref, o_ref):
    x = x_ref[...]
    h = jnp.dot(x, w1t_ref[...], preferred_element_type=jnp.float32)
    h = jnp.maximum(h + b1_ref[...], 0.0)
    o = jnp.dot(h, w2_ref[...], preferred_element_type=jnp.float32)
    o_ref[...] = o + b2_ref[0, 0]


def _mlp(combined, w1t, b1_2d, w2t, b2_2d):
    BT = 2048
    grid = (B // BT,)
    return pl.pallas_call(
        _mlp_body,
        grid=grid,
        in_specs=[
            pl.BlockSpec((BT, D + 1), lambda i: (i, 0)),
            pl.BlockSpec((D + 1, HIDDEN), lambda i: (0, 0)),
            pl.BlockSpec((1, HIDDEN), lambda i: (0, 0)),
            pl.BlockSpec((HIDDEN, 1), lambda i: (0, 0)),
            pl.BlockSpec((1, 1), lambda i: (0, 0)),
        ],
        out_specs=pl.BlockSpec((BT, 1), lambda i: (i, 0)),
        out_shape=jax.ShapeDtypeStruct((B, 1), jnp.float32),
    )(combined, w1t, b1_2d, w2t, b2_2d)


def kernel(lineup, table, W1, b1, W2, b2):
    idx3 = (
        lineup[:, :S]
        .reshape(NW, NCHUNK, CHUNK, S)
        .transpose(0, 1, 3, 2)
        .reshape(NW, NCHUNK * S, CHUNK)
    )
    summed = _gather_sum(idx3, table)

    flag = lineup[:, S:].astype(jnp.float32)
    combined = jnp.concatenate([summed, flag], axis=1)
    return _mlp(combined, W1.T, b1.reshape(1, HIDDEN), W2.T, b2.reshape(1, 1))


# speedup vs baseline: 1.7215x; 1.0094x over previous
import jax
import jax.numpy as jnp
from jax import lax
from jax.experimental import pallas as pl
from jax.experimental.pallas import tpu as pltpu
from jax.experimental.pallas import tpu_sc as plsc

B = 16384
L = 21
S = 20
D = 32
HIDDEN = 256

NC, NS = 2, 16
NW = NC * NS
BPW = B // NW
CHUNK = 128
NCHUNK = BPW // CHUNK


def _gather_sum_body(lineup_hbm, table_hbm, out_hbm, lin_v, idx_v, acc_v, sem):
    wid = lax.axis_index("s") * NC + lax.axis_index("c")
    base = wid * BPW

    pltpu.sync_copy(lineup_hbm.at[pl.ds(base * L, BPW * L)], lin_v)

    zero = jnp.zeros((16,), jnp.float32)

    def zero_body(i, carry):
        for u in range(4):
            r = i * 4 + u
            acc_v[r, pl.ds(0, 16)] = zero
            acc_v[r, pl.ds(16, 16)] = zero
        return carry

    lax.fori_loop(0, BPW // 4, zero_body, 0)

    lane = lax.iota(jnp.int32, 16)

    def build_body(t, carry):
        c = t // S
        j = t % S
        rowbase = c * CHUNK
        for g in range(CHUNK // 16):
            flat = (rowbase + g * 16 + lane) * L + j
            idx_v[t, pl.ds(g * 16, 16)] = plsc.load_gather(lin_v, [flat])
        return carry

    lax.fori_loop(0, NCHUNK * S, build_body, 0)

    def chunk_body(c, carry):
        dst = acc_v.at[pl.ds(c * CHUNK, CHUNK)]
        descs = []
        for j in range(S):
            d = pltpu.async_copy(
                table_hbm.at[idx_v.at[c * S + j]], dst, sem, add=True
            )
            descs.append(d)
        for d in descs:
            d.wait()
        return carry

    lax.fori_loop(0, NCHUNK, chunk_body, 0)

    pltpu.sync_copy(acc_v, out_hbm.at[pl.ds(base, BPW)])


def _gather_sum(lineup_flat, table):
    mesh = plsc.VectorSubcoreMesh(
        core_axis_name="c", subcore_axis_name="s", num_cores=NC, num_subcores=NS
    )
    return pl.kernel(
        _gather_sum_body,
        out_type=jax.ShapeDtypeStruct((B, D), jnp.float32),
        mesh=mesh,
        scratch_types=[
            pltpu.VMEM((BPW * L,), jnp.int32),
            pltpu.VMEM((NCHUNK * S, CHUNK), jnp.int32),
            pltpu.VMEM((BPW, D), jnp.float32),
            pltpu.SemaphoreType.DMA,
        ],
        compiler_params=pltpu.CompilerParams(
            use_tc_tiling_on_sc=False, needs_layout_passes=False
        ),
    )(lineup_flat, table)


def _mlp_body(x_ref, lineup_ref, w1t_ref, b1_ref, w2_ref, b2_ref, o_ref):
    x = x_ref[...]
    flag = lineup_ref[:, S:].astype(jnp.float32)
    x33 = jnp.concatenate([x, flag], axis=1)
    h = jnp.dot(x33, w1t_ref[...], preferred_element_type=jnp.float32)
    h = jnp.maximum(h + b1_ref[...], 0.0)
    o = jnp.dot(h, w2_ref[...], preferred_element_type=jnp.float32)
    o_ref[...] = o + b2_ref[0, 0]


def _mlp(summed, lineup, w1t, b1_2d, w2t, b2_2d):
    BT = 2048
    grid = (B // BT,)
    return pl.pallas_call(
        _mlp_body,
        grid=grid,
        in_specs=[
            pl.BlockSpec((BT, D), lambda i: (i, 0)),
            pl.BlockSpec((BT, L), lambda i: (i, 0)),
            pl.BlockSpec((D + 1, HIDDEN), lambda i: (0, 0)),
            pl.BlockSpec((1, HIDDEN), lambda i: (0, 0)),
            pl.BlockSpec((HIDDEN, 1), lambda i: (0, 0)),
            pl.BlockSpec((1, 1), lambda i: (0, 0)),
        ],
        out_specs=pl.BlockSpec((BT, 1), lambda i: (i, 0)),
        out_shape=jax.ShapeDtypeStruct((B, 1), jnp.float32),
    )(summed, lineup, w1t, b1_2d, w2t, b2_2d)


def kernel(lineup, table, W1, b1, W2, b2):
    summed = _gather_sum(lineup.reshape(-1), table)
    return _mlp(summed, lineup, W1.T, b1.reshape(1, HIDDEN), W2.T,
                b2.reshape(1, 1))
